# Initial kernel scaffold; baseline (speedup 1.0000x reference)
import jax
import jax.numpy as jnp
from jax.experimental import pallas as pl


def kernel(x, w3, b3, w1, b1, w2, b2, wr, br):
    raise NotImplementedError("write your pallas kernel here")


# speedup vs baseline: 1.4577x; 1.4577x over previous
import functools

import jax
import jax.numpy as jnp
from jax import lax
from jax.experimental import pallas as pl
from jax.experimental.pallas import tpu as pltpu


def _gelu_exact(x):
    return 0.5 * x * (1.0 + lax.erf(x * 0.7071067811865476))


def _sab_kernel(x_ref, wf_ref, bf_ref, w2m_ref, b3b_ref, b2_ref,
                wr_ref, br_ref, o_ref, *, kcout):
    x = x_ref[...].astype(jnp.bfloat16)
    g = jnp.dot(x, wf_ref[...], preferred_element_type=jnp.float32)
    g = _gelu_exact(g + bf_ref[...])
    o2 = jnp.dot(g.astype(jnp.bfloat16), w2m_ref[...],
                 preferred_element_type=jnp.float32)
    out = o2[:, :kcout] + b3b_ref[...]
    r = jax.nn.sigmoid(o2[:, kcout:2 * kcout] + b2_ref[...])
    r = r * wr_ref[...] + br_ref[...]
    o_ref[...] = (r * out).astype(o_ref.dtype)


def _sab_rows(x_rows, p, *, tile=1024):
    rows, cin = x_rows.shape
    cout = p["w3"].shape[1]
    cmid = p["w1"].shape[1]

    if cout < 128 and 128 % cout == 0 and rows % (128 // cout) == 0:
        k = 128 // cout
    else:
        k = 1
    kcin, kcout, kcmid = k * cin, k * cout, k * cmid
    prows = rows // k

    eye = jnp.eye(k, dtype=jnp.float32)
    w3b = jnp.kron(eye, p["w3"])
    w1b = jnp.kron(eye, p["w1"])
    npad = max(0, 2 * kcout - (kcout + kcmid))
    wf = jnp.concatenate(
        [w3b, w1b, jnp.zeros((kcin, npad), jnp.float32)], axis=1)
    bf = jnp.concatenate(
        [jnp.tile(p["b3"], (1, k)), jnp.tile(p["b1"], (1, k)),
         jnp.zeros((1, npad), jnp.float32)], axis=1)

    s_blk = jnp.tile(jnp.transpose(p["w2"]), (1, cout))
    s_bcast = jnp.kron(eye, s_blk)
    zc = jnp.zeros((kcout, kcout), jnp.float32)
    top = jnp.concatenate([w3b, zc], axis=1)
    mid = jnp.concatenate(
        [jnp.zeros((kcmid, kcout), jnp.float32), s_bcast], axis=1)
    bot = jnp.zeros((npad, 2 * kcout), jnp.float32)
    w2m = jnp.concatenate([top, mid, bot], axis=0)

    b3b = jnp.tile(p["b3"], (1, k))
    wrt = jnp.tile(p["wr"], (1, k))
    brt = jnp.tile(p["br"], (1, k))

    x_packed = x_rows.reshape(prows, kcin)
    tile = min(tile, prows)
    tile = max(8, (tile // 8) * 8)
    nsteps = pl.cdiv(prows, tile)

    wf_b = wf.astype(jnp.bfloat16)
    w2m_b = w2m.astype(jnp.bfloat16)

    full = lambda i: (0, 0)
    out = pl.pallas_call(
        functools.partial(_sab_kernel, kcout=kcout),
        out_shape=jax.ShapeDtypeStruct((nsteps * tile, kcout), x_rows.dtype),
        grid_spec=pltpu.PrefetchScalarGridSpec(
            num_scalar_prefetch=0,
            grid=(nsteps,),
            in_specs=[
                pl.BlockSpec((tile, kcin), lambda i: (i, 0)),
                pl.BlockSpec(wf_b.shape, full),
                pl.BlockSpec(bf.shape, full),
                pl.BlockSpec(w2m_b.shape, full),
                pl.BlockSpec(b3b.shape, full),
                pl.BlockSpec(p["b2"].shape, full),
                pl.BlockSpec(wrt.shape, full),
                pl.BlockSpec(brt.shape, full),
            ],
            out_specs=pl.BlockSpec((tile, kcout), lambda i: (i, 0)),
        ),
        compiler_params=pltpu.CompilerParams(
            dimension_semantics=("parallel",),
            vmem_limit_bytes=64 * 1024 * 1024),
    )(x_packed, wf_b, bf, w2m_b, b3b, p["b2"], wrt, brt)

    return out[:prows].reshape(rows, cout)


def kernel(x, w3, b3, w1, b1, w2, b2, wr, br):
    p = {"w3": w3, "b3": b3, "w1": w1, "b1": b1,
         "w2": w2, "b2": b2, "wr": wr, "br": br}
    n, h, w, c = x.shape
    cout = w3.shape[1]
    y = _sab_rows(x.reshape(n * h * w, c), p)
    return y.reshape(n, h, w, cout)


# speedup vs baseline: 1.7844x; 1.2241x over previous
import functools

import jax
import jax.numpy as jnp
from jax import lax
from jax.experimental import pallas as pl
from jax.experimental.pallas import tpu as pltpu


def _gelu_exact(x):
    return 0.5 * x * (1.0 + lax.erf(x * 0.7071067811865476))


def _sab_kernel(x_ref, wft_ref, bft_ref, w2mt_ref, b3t_ref, b2_ref,
                wrt_ref, brt_ref, eye_ref, o_ref, *, cin, cout):
    blk = x_ref.shape
    px = blk[0] * blk[1] * blk[2]
    x = x_ref[...].reshape(px, cin).astype(jnp.bfloat16)

    gt = lax.dot_general(wft_ref[...], x, (((1,), (1,)), ((), ())),
                         preferred_element_type=jnp.float32)
    gt = _gelu_exact(gt + bft_ref[...]).astype(jnp.bfloat16)

    o2t = jnp.dot(w2mt_ref[...], gt, preferred_element_type=jnp.float32)

    out = o2t[:cout] + b3t_ref[...]
    s = jax.nn.sigmoid(o2t[cout:cout + 1] + b2_ref[...])
    r = s * wrt_ref[...] + brt_ref[...]
    yt = (r * out).astype(jnp.bfloat16)

    y = lax.dot_general(yt, eye_ref[...], (((0,), (0,)), ((), ())),
                        preferred_element_type=jnp.float32)
    o_ref[...] = y.reshape(blk).astype(o_ref.dtype)


def kernel(x, w3, b3, w1, b1, w2, b2, wr, br):
    n, h, w, cin = x.shape
    cout = w3.shape[1]
    cmid = w1.shape[1]
    m = cin + cmid
    mp = ((m + 1 + 7) // 8) * 8

    wft = jnp.concatenate(
        [w3.T, w1.T, jnp.zeros((mp - m, cin), jnp.float32)], axis=0)
    bft = jnp.concatenate(
        [b3.T, b1.T, jnp.zeros((mp - m, 1), jnp.float32)], axis=0)
    w2mt = jnp.zeros((mp, mp), jnp.float32)
    w2mt = w2mt.at[:cout, :cout].set(w3.T)
    w2mt = w2mt.at[cout, cin:cin + cmid].set(w2[0])

    wft_b = wft.astype(jnp.bfloat16)
    w2mt_b = w2mt.astype(jnp.bfloat16)
    eye = jnp.eye(cout, dtype=jnp.bfloat16)
    b3t, wrt, brt = b3.T, wr.T, br.T

    hb = 64 if h % 64 == 0 else h
    grid = (n, h // hb)
    full = lambda i, j: (0, 0)
    y = pl.pallas_call(
        functools.partial(_sab_kernel, cin=cin, cout=cout),
        out_shape=jax.ShapeDtypeStruct((n, h, w, cout), x.dtype),
        grid_spec=pltpu.PrefetchScalarGridSpec(
            num_scalar_prefetch=0,
            grid=grid,
            in_specs=[
                pl.BlockSpec((1, hb, w, cin), lambda i, j: (i, j, 0, 0)),
                pl.BlockSpec(wft_b.shape, full),
                pl.BlockSpec(bft.shape, full),
                pl.BlockSpec(w2mt_b.shape, full),
                pl.BlockSpec(b3t.shape, full),
                pl.BlockSpec(b2.shape, full),
                pl.BlockSpec(wrt.shape, full),
                pl.BlockSpec(brt.shape, full),
                pl.BlockSpec(eye.shape, full),
            ],
            out_specs=pl.BlockSpec((1, hb, w, cout), lambda i, j: (i, j, 0, 0)),
        ),
        compiler_params=pltpu.CompilerParams(
            dimension_semantics=("parallel", "parallel"),
            vmem_limit_bytes=64 * 1024 * 1024),
    )(x, wft_b, bft, w2mt_b, b3t, b2, wrt, brt, eye)
    return y


# speedup vs baseline: 2.5331x; 1.4196x over previous
import functools

import jax
import jax.numpy as jnp
from jax import lax
from jax.experimental import pallas as pl
from jax.experimental.pallas import tpu as pltpu


def _gelu_exact(x):
    return 0.5 * x * (1.0 + lax.erf(x * 0.7071067811865476))


def _sab_kernel(x_ref, wf_ref, bf_ref, w2m_ref, b3b_ref, b2_ref,
                wr_ref, br_ref, o_ref, *, kcout):
    blk = x_ref.shape
    rows = blk[0] * blk[1] * blk[2]
    x = x_ref[...].reshape(rows, blk[3]).astype(jnp.bfloat16)
    g = jnp.dot(x, wf_ref[...], preferred_element_type=jnp.float32)
    g = _gelu_exact(g + bf_ref[...])
    o2 = jnp.dot(g.astype(jnp.bfloat16), w2m_ref[...],
                 preferred_element_type=jnp.float32)
    out = o2[:, :kcout] + b3b_ref[...]
    r = jax.nn.sigmoid(o2[:, kcout:2 * kcout] + b2_ref[...])
    r = r * wr_ref[...] + br_ref[...]
    o_ref[...] = (r * out).astype(o_ref.dtype).reshape(blk)


def kernel(x, w3, b3, w1, b1, w2, b2, wr, br):
    n, h, w, cin = x.shape
    cout = w3.shape[1]
    cmid = w1.shape[1]
    k = 128 // cout
    kcin, kcout, kcmid = k * cin, k * cout, k * cmid

    eye = jnp.eye(k, dtype=jnp.float32)
    w3b = jnp.kron(eye, w3)
    w1b = jnp.kron(eye, w1)
    npad = kcout - kcmid
    wf = jnp.concatenate(
        [w3b, w1b, jnp.zeros((kcin, npad), jnp.float32)], axis=1)
    bf = jnp.concatenate(
        [jnp.tile(b3, (1, k)), jnp.tile(b1, (1, k)),
         jnp.zeros((1, npad), jnp.float32)], axis=1)

    s_blk = jnp.tile(jnp.transpose(w2), (1, cout))
    s_bcast = jnp.kron(eye, s_blk)
    w2m = jnp.concatenate([
        jnp.concatenate([w3b, jnp.zeros((kcin, kcout), jnp.float32)], axis=1),
        jnp.concatenate([jnp.zeros((kcmid, kcout), jnp.float32), s_bcast],
                        axis=1),
        jnp.zeros((npad, 2 * kcout), jnp.float32)], axis=0)

    b3b = jnp.tile(b3, (1, k))
    wrt = jnp.tile(wr, (1, k))
    brt = jnp.tile(br, (1, k))
    wf_b = wf.astype(jnp.bfloat16)
    w2m_b = w2m.astype(jnp.bfloat16)

    xv = x.reshape(n, h, w // k, kcin)

    hb = 64 if h % 64 == 0 else h
    grid = (n, h // hb)
    full = lambda i, j: (0, 0)
    y = pl.pallas_call(
        functools.partial(_sab_kernel, kcout=kcout),
        out_shape=jax.ShapeDtypeStruct((n, h, w // k, kcout), x.dtype),
        grid_spec=pltpu.PrefetchScalarGridSpec(
            num_scalar_prefetch=0,
            grid=grid,
            in_specs=[
                pl.BlockSpec((1, hb, w // k, kcin), lambda i, j: (i, j, 0, 0)),
                pl.BlockSpec(wf_b.shape, full),
                pl.BlockSpec(bf.shape, full),
                pl.BlockSpec(w2m_b.shape, full),
                pl.BlockSpec(b3b.shape, full),
                pl.BlockSpec(b2.shape, full),
                pl.BlockSpec(wrt.shape, full),
                pl.BlockSpec(brt.shape, full),
            ],
            out_specs=pl.BlockSpec((1, hb, w // k, kcout),
                                   lambda i, j: (i, j, 0, 0)),
        ),
        compiler_params=pltpu.CompilerParams(
            dimension_semantics=("parallel", "parallel"),
            vmem_limit_bytes=64 * 1024 * 1024),
    )(xv, wf_b, bf, w2m_b, b3b, b2, wrt, brt)
    return y.reshape(n, h, w, cout)


# speedup vs baseline: 5.7881x; 2.2850x over previous
import functools

import jax
import jax.numpy as jnp
from jax import lax
from jax.experimental import pallas as pl
from jax.experimental.pallas import tpu as pltpu


def _gelu_exact(x):
    return 0.5 * x * (1.0 + lax.erf(x * 0.7071067811865476))


def _sab_kernel(x_ref, wm_ref, wa_ref, ws_ref, bm_ref, ba_ref, b2_ref,
                wr_ref, br_ref, o_ref, *, cin, cmid, rt, ngroups):
    gr = rt * cin
    wlanes = x_ref.shape[1]
    wm = wm_ref[...]
    wa = wa_ref[...]
    ws = ws_ref[...]
    for g in range(ngroups):
        xg = x_ref[g * gr:(g + 1) * gr, :].astype(jnp.bfloat16)
        hm = _gelu_exact(
            jnp.dot(wm, xg, preferred_element_type=jnp.float32) + bm_ref[...])
        ha = _gelu_exact(
            jnp.dot(wa, xg, preferred_element_type=jnp.float32) + ba_ref[...])
        outm = jnp.dot(wm, hm.astype(jnp.bfloat16),
                       preferred_element_type=jnp.float32) + bm_ref[...]
        s = jax.nn.sigmoid(
            jnp.dot(ws, ha.astype(jnp.bfloat16),
                    preferred_element_type=jnp.float32) + b2_ref[...])
        sb = jnp.broadcast_to(
            s.reshape(rt, 1, wlanes), (rt, cin, wlanes)).reshape(gr, wlanes)
        r = sb * wr_ref[...] + br_ref[...]
        o_ref[g * gr:(g + 1) * gr, :] = (r * outm).astype(o_ref.dtype)


def kernel(x, w3, b3, w1, b1, w2, b2, wr, br):
    n, h, w, cin = x.shape
    cout = w3.shape[1]
    cmid = w1.shape[1]
    rt = 256 // cin

    eye = jnp.eye(rt, dtype=jnp.float32)
    wm = jnp.kron(eye, w3.T).astype(jnp.bfloat16)
    wa = jnp.kron(eye, w1.T).astype(jnp.bfloat16)
    ws = jnp.kron(eye, w2).astype(jnp.bfloat16)
    bm = jnp.tile(b3.T, (rt, 1))
    ba = jnp.tile(b1.T, (rt, 1))
    wrb = jnp.tile(wr.T, (rt, 1))
    brb = jnp.tile(br.T, (rt, 1))

    x2 = jnp.transpose(x, (0, 1, 3, 2)).reshape(n * h * cin, w)

    slices = n * h
    sl_per_step = min(64, slices)
    rows = sl_per_step * cin
    grid = (slices // sl_per_step,)
    ngroups = sl_per_step // rt
    full = lambda i: (0, 0)
    y = pl.pallas_call(
        functools.partial(_sab_kernel, cin=cin, cmid=cmid, rt=rt,
                          ngroups=ngroups),
        out_shape=jax.ShapeDtypeStruct((slices * cout, w), x.dtype),
        grid_spec=pltpu.PrefetchScalarGridSpec(
            num_scalar_prefetch=0,
            grid=grid,
            in_specs=[
                pl.BlockSpec((rows, w), lambda i: (i, 0)),
                pl.BlockSpec(wm.shape, full),
                pl.BlockSpec(wa.shape, full),
                pl.BlockSpec(ws.shape, full),
                pl.BlockSpec(bm.shape, full),
                pl.BlockSpec(ba.shape, full),
                pl.BlockSpec(b2.shape, full),
                pl.BlockSpec(wrb.shape, full),
                pl.BlockSpec(brb.shape, full),
            ],
            out_specs=pl.BlockSpec((sl_per_step * cout, w), lambda i: (i, 0)),
        ),
        compiler_params=pltpu.CompilerParams(
            dimension_semantics=("parallel",),
            vmem_limit_bytes=64 * 1024 * 1024),
    )(x2, wm, wa, ws, bm, ba, b2, wrb, brb)
    return jnp.transpose(y.reshape(n, h, cout, w), (0, 1, 3, 2))
